# baseline (speedup 1.0000x reference)
import jax
import jax.numpy as jnp
import numpy as np
from jax import lax
from jax.experimental import pallas as pl
from jax.experimental.pallas import tpu as pltpu
from jax.experimental.pallas import tpu_sc as plsc

N = 10000
E = 160000
D = 128
H = 8
DH = 16
EPS = 1e-8

_BLK = 1000
_NBLK = N // _BLK

NW = 32
EPAD = 163840
PADN = 10008
CH1 = 16
PW1 = EPAD // CH1 // NW
CH2 = 16
PW2 = EPAD // CH2 // NW
ROWS_T = 624
ND = 640
DROWS_T = ND // 16
NPAD = 10016


def _proj_body(h_ref, qw_ref, qb_ref, kw_ref, kb_ref, vw_ref, vb_ref,
               bd0_ref, bd1_ref, qs_ref, kt0_ref, kt1_ref, v_ref):
    hb = h_ref[...]
    k = hb @ kw_ref[...] + kb_ref[...]
    qs_ref[...] = hb @ qw_ref[...] + qb_ref[...]
    kt0_ref[...] = k @ bd0_ref[...]
    kt1_ref[...] = k @ bd1_ref[...]
    v_ref[...] = hb @ vw_ref[...] + vb_ref[...]


def _rcp_body(gl_ref, dn0_ref, dn1_ref, rcp0_ref, rcp1_ref):
    g = jax.nn.sigmoid(gl_ref[...])
    d0 = dn0_ref[...]
    d1 = dn1_ref[...]
    rcp0_ref[...] = g[0, 0] / jnp.maximum(d0[0] + d0[1], EPS)
    rcp1_ref[...] = g[0, 1] / jnp.maximum(d1[0] + d1[1], EPS)


def _out_body(hmsgp_ref, h_ref, ow_ref, ob_ref, lns_ref, lnb_ref, o_ref):
    hm = hmsgp_ref[...]
    t = jax.nn.gelu(hm[0] + hm[1]) @ ow_ref[...] + ob_ref[...]
    x = t + h_ref[...]
    mu = jnp.mean(x, axis=-1, keepdims=True)
    var = jnp.mean((x - mu) ** 2, axis=-1, keepdims=True)
    o_ref[...] = (x - mu) / jnp.sqrt(var + 1e-5) * lns_ref[...] + lnb_ref[...]


def _worker_ids():
    c = lax.axis_index("c")
    s = lax.axis_index("s")
    return c, s, s * 2 + c


def _lanes():
    return lax.iota(jnp.int32, 16)


def _p1_compute(idxb, qg, ktg, exv, ex128):
    def group(g, _):
        rows = g * 16 + _lanes()
        colb = (idxb[1, pl.ds(g * 16, 16)] & 15) * 8
        for h in range(H):
            acc = None
            for dd in range(DH):
                col = jnp.full((16,), h * DH + dd, jnp.int32)
                qv = plsc.load_gather(qg, [rows, col])
                kv = plsc.load_gather(ktg, [rows, col])
                acc = qv * kv if acc is None else acc + qv * kv
            ex = jnp.exp(acc)
            plsc.store_scatter(exv, [rows, jnp.full((16,), h, jnp.int32)], ex)
            plsc.store_scatter(ex128, [rows, colb + h], ex)
        return 0

    lax.fori_loop(0, CH1 // 16, group, 0)


def _p1_rezero(idxb, ex128):
    def group(g, _):
        rows = g * 16 + _lanes()
        colb = (idxb[1, pl.ds(g * 16, 16)] & 15) * 8
        zv = jnp.zeros((16,), jnp.float32)
        for h in range(H):
            plsc.store_scatter(ex128, [rows, colb + h], zv)
        return 0

    lax.fori_loop(0, CH1 // 16, group, 0)


def _sc_phase1(pidx0, pidx1, qs, kt0, kt1, zn,
               ex0, ex1, dnp0, dnp1,
               idxb, qg, ktg, exv, ex128, dn0_sh, dn1_sh,
               sem_i, sem_gq, sem_gk, sem_s, sem_w):
    sc, tid, wid = _worker_ids()
    drb = tid * DROWS_T
    stg = ex128[0].at[pl.ds(0, 16)]

    def zchunk(j, _):
        rb = drb + j * 8
        s8 = ex128[0].at[pl.ds(0, 8)]
        pltpu.sync_copy(zn.at[pl.ds(rb, 8)], s8)
        pltpu.sync_copy(s8, dn0_sh.at[pl.ds(rb, 8)])
        pltpu.sync_copy(s8, dn1_sh.at[pl.ds(rb, 8)])
        return 0

    lax.fori_loop(0, DROWS_T // 8, zchunk, 0)
    plsc.subcore_barrier()

    cb0 = wid * PW1

    for pidx, kt, exh, dnsh in ((pidx0, kt0, ex0, dn0_sh),
                                (pidx1, kt1, ex1, dn1_sh)):
        pltpu.sync_copy(zn.at[pl.ds(0, CH1)], ex128[0])
        pltpu.sync_copy(zn.at[pl.ds(0, CH1)], ex128[1])
        for r in range(3):
            pltpu.async_copy(pidx.at[cb0 + r], idxb[r], sem_i[r])
        pltpu.make_async_copy(pidx.at[cb0], idxb[0], sem_i[0]).wait()
        pltpu.async_copy(qs.at[idxb[0].at[1]], qg[0], sem_gq[0])
        pltpu.async_copy(kt.at[idxb[0].at[0]], ktg[0], sem_gk[0])

        def quad(j, _, pidx=pidx, kt=kt, exh=exh, dnsh=dnsh):
            for r in range(4):
                i = j * 4 + r
                b = r % 2
                rn = (r + 1) % 4
                ro = (r - 1) % 4

                @pl.when(i + 1 < PW1)
                def _(i=i, b=b, rn=rn):
                    pltpu.make_async_copy(pidx.at[cb0 + i + 1], idxb[rn],
                                          sem_i[rn]).wait()
                    pltpu.async_copy(qs.at[idxb[rn].at[1]], qg[1 - b],
                                     sem_gq[1 - b])
                    pltpu.async_copy(kt.at[idxb[rn].at[0]], ktg[1 - b],
                                     sem_gk[1 - b])

                pltpu.make_async_copy(qs.at[idxb[r].at[1]], qg[b],
                                      sem_gq[b]).wait()
                pltpu.make_async_copy(kt.at[idxb[r].at[0]], ktg[b],
                                      sem_gk[b]).wait()
                _p1_compute(idxb[r], qg[b], ktg[b], exv[b], ex128[b])
                pltpu.async_copy(ex128[b], dnsh.at[idxb[r].at[2]], sem_s[b],
                                 add=True)
                pltpu.async_copy(exv[b], exh.at[pl.ds((cb0 + i) * CH1, CH1)],
                                 sem_w[b])

                @pl.when(i >= 1)
                def _(i=i, b=b, ro=ro, dnsh=dnsh, exh=exh):
                    pltpu.make_async_copy(ex128[1 - b], dnsh.at[idxb[ro].at[2]],
                                          sem_s[1 - b]).wait()
                    pltpu.make_async_copy(
                        exv[1 - b], exh.at[pl.ds((cb0 + i - 1) * CH1, CH1)],
                        sem_w[1 - b]).wait()
                    _p1_rezero(idxb[ro], ex128[1 - b])

                @pl.when(i + 3 < PW1)
                def _(i=i, ro=ro, pidx=pidx):
                    pltpu.async_copy(pidx.at[cb0 + i + 3], idxb[ro], sem_i[ro])
            return 0

        lax.fori_loop(0, PW1 // 4, quad, 0)
        bl = (PW1 - 1) % 2
        rl = (PW1 - 1) % 4
        pltpu.make_async_copy(ex128[bl], dnsh.at[idxb[rl].at[2]],
                              sem_s[bl]).wait()
        pltpu.make_async_copy(exv[bl], exh.at[pl.ds((cb0 + PW1 - 1) * CH1, CH1)],
                              sem_w[bl]).wait()
        _p1_rezero(idxb[rl], ex128[bl])

    plsc.subcore_barrier()

    def rchunk(j, _):
        rb = drb + j * 8
        s8 = ex128[0].at[pl.ds(0, 8)]
        pltpu.sync_copy(dn0_sh.at[pl.ds(rb, 8)], s8)
        pltpu.sync_copy(s8, dnp0.at[sc, pl.ds(rb, 8)])
        pltpu.sync_copy(dn1_sh.at[pl.ds(rb, 8)], s8)
        pltpu.sync_copy(s8, dnp1.at[sc, pl.ds(rb, 8)])
        return 0

    lax.fori_loop(0, DROWS_T // 8, rchunk, 0)


def _p2_compute(idxb, vg, rcpg, exv, msgv):
    def group(g, _):
        rows = g * 16 + _lanes()
        colb = (idxb[1, pl.ds(g * 16, 16)] & 15) * 8
        for h in range(H):
            aw = (plsc.load_gather(exv, [rows, jnp.full((16,), h, jnp.int32)]) *
                  plsc.load_gather(rcpg, [rows, colb + h]))
            for dd in range(DH):
                col = jnp.full((16,), h * DH + dd, jnp.int32)
                mv = plsc.load_gather(vg, [rows, col]) * aw
                plsc.store_scatter(msgv, [rows, col], mv)
        return 0

    lax.fori_loop(0, CH2 // 16, group, 0)


def _sc_phase2(pidx0, pidx1, v, ex0, ex1, rcp0, rcp1, zn,
               hmsgp,
               idxb, vg, rcpg, exv, msgv, hmsg_sh, rcp_sh,
               sem_i, sem_gv, sem_gr, sem_ge, sem_s):
    sc, tid, wid = _worker_ids()
    rbase = tid * ROWS_T
    nzc = jnp.where(tid == 15, 40, 39)
    stg = msgv[0].at[pl.ds(0, 16)]

    def zchunk(j, _):
        rb = rbase + j * 16
        pltpu.sync_copy(zn.at[pl.ds(rb, 16)], stg)
        pltpu.sync_copy(stg, hmsg_sh.at[pl.ds(rb, 16)])
        return 0

    lax.fori_loop(0, nzc, zchunk, 0)

    s8 = msgv[1].at[pl.ds(0, 8)]

    def pchunk(j, _):
        rb = tid * (ND // 16) + j * 8
        pltpu.sync_copy(rcp0.at[pl.ds(rb, 8)], s8)
        pltpu.sync_copy(s8, rcp_sh.at[pl.ds(rb, 8)])
        pltpu.sync_copy(rcp1.at[pl.ds(rb, 8)], s8)
        pltpu.sync_copy(s8, rcp_sh.at[pl.ds(ND + rb, 8)])
        return 0

    lax.fori_loop(0, ND // 16 // 8, pchunk, 0)
    plsc.subcore_barrier()

    cb0 = wid * PW2

    for pidx, exh in ((pidx0, ex0), (pidx1, ex1)):
        for r in range(3):
            pltpu.async_copy(pidx.at[cb0 + r], idxb[r], sem_i[r])
        pltpu.make_async_copy(pidx.at[cb0], idxb[0], sem_i[0]).wait()
        pltpu.async_copy(v.at[idxb[0].at[0]], vg[0], sem_gv[0])
        pltpu.async_copy(rcp_sh.at[idxb[0].at[2]], rcpg[0], sem_gr[0])
        pltpu.async_copy(exh.at[pl.ds(cb0 * CH2, CH2)], exv[0], sem_ge[0])

        def quad(j, _, pidx=pidx, exh=exh):
            for r in range(4):
                i = j * 4 + r
                b = r % 2
                rn = (r + 1) % 4
                ro = (r - 1) % 4

                @pl.when(i + 1 < PW2)
                def _(i=i, b=b, rn=rn, exh=exh):
                    pltpu.make_async_copy(pidx.at[cb0 + i + 1], idxb[rn],
                                          sem_i[rn]).wait()
                    pltpu.async_copy(v.at[idxb[rn].at[0]], vg[1 - b],
                                     sem_gv[1 - b])
                    pltpu.async_copy(rcp_sh.at[idxb[rn].at[2]], rcpg[1 - b],
                                     sem_gr[1 - b])
                    pltpu.async_copy(exh.at[pl.ds((cb0 + i + 1) * CH2, CH2)],
                                     exv[1 - b], sem_ge[1 - b])

                pltpu.make_async_copy(v.at[idxb[r].at[0]], vg[b],
                                      sem_gv[b]).wait()
                pltpu.make_async_copy(rcp_sh.at[idxb[r].at[2]], rcpg[b],
                                      sem_gr[b]).wait()
                pltpu.make_async_copy(exh.at[pl.ds((cb0 + i) * CH2, CH2)],
                                      exv[b], sem_ge[b]).wait()
                _p2_compute(idxb[r], vg[b], rcpg[b], exv[b], msgv[b])
                pltpu.async_copy(msgv[b], hmsg_sh.at[idxb[r].at[3]], sem_s[b],
                                 add=True)

                @pl.when(i >= 1)
                def _(i=i, b=b, ro=ro):
                    pltpu.make_async_copy(msgv[1 - b],
                                          hmsg_sh.at[idxb[ro].at[3]],
                                          sem_s[1 - b]).wait()

                @pl.when(i + 3 < PW2)
                def _(i=i, ro=ro, pidx=pidx):
                    pltpu.async_copy(pidx.at[cb0 + i + 3], idxb[ro], sem_i[ro])
            return 0

        lax.fori_loop(0, PW2 // 4, quad, 0)
        bl = (PW2 - 1) % 2
        rl = (PW2 - 1) % 4
        pltpu.make_async_copy(msgv[bl], hmsg_sh.at[idxb[rl].at[3]],
                              sem_s[bl]).wait()

    plsc.subcore_barrier()

    def rchunk(j, _):
        rb = rbase + j * 16
        pltpu.sync_copy(hmsg_sh.at[pl.ds(rb, 16)], stg)
        pltpu.sync_copy(stg, hmsgp.at[sc, pl.ds(rb, 16)])
        return 0

    lax.fori_loop(0, nzc, rchunk, 0)


def kernel(h, src_idx0, dst_idx0, src_idx1, dst_idx1, Qw, Qb, Kw, Kb, Vw, Vb,
           edge_W, gate_logits, Ow, Ob, ln_s, ln_b):
    scale = 1.0 / np.sqrt(DH)
    bd = jnp.zeros((2, H, DH, H, DH), jnp.float32)
    bd = bd.at[:, jnp.arange(H), :, jnp.arange(H), :].set(
        jnp.transpose(edge_W, (1, 0, 2, 3))).reshape(2, D, D)

    row = lambda b: b.reshape(1, D)
    wspec = pl.BlockSpec((D, D), lambda i: (0, 0))
    bspec = pl.BlockSpec((1, D), lambda i: (0, 0))
    nspec = pl.BlockSpec((_BLK, D), lambda i: (i, 0))
    nshape = jax.ShapeDtypeStruct((N, D), jnp.float32)

    qs, kt0, kt1, v = pl.pallas_call(
        _proj_body,
        grid=(_NBLK,),
        in_specs=[nspec, wspec, bspec, wspec, bspec, wspec, bspec, wspec, wspec],
        out_specs=[nspec, nspec, nspec, nspec],
        out_shape=[nshape, nshape, nshape, nshape],
    )(h, Qw * scale, row(Qb) * scale, Kw, row(Kb), Vw, row(Vb), bd[0], bd[1])

    mesh = plsc.VectorSubcoreMesh(core_axis_name="c", subcore_axis_name="s")
    f32 = jnp.float32
    i32 = jnp.int32
    sc_params = pltpu.CompilerParams(needs_layout_passes=False)
    zn = jnp.zeros((N, D), f32)

    npad = EPAD - E

    def pack_idx(si, di, ch, dn_off=0):
        zp = jnp.zeros((npad,), i32)
        si_p = jnp.concatenate([si, zp])
        dig = jnp.concatenate([di, zp])
        dnr = jnp.concatenate([lax.shift_right_logical(di, 4) + dn_off,
                               jnp.full((npad,), dn_off + ND - 1, i32)])
        hmr = jnp.concatenate([di, jnp.full((npad,), PADN, i32)])
        nch = EPAD // ch
        z = jnp.zeros((nch, ch), i32)
        return jnp.stack([a.reshape(nch, ch) for a in (si_p, dig, dnr, hmr)] +
                         [z, z, z, z], axis=1)

    p1i0 = pack_idx(src_idx0, dst_idx0, CH1)
    p1i1 = pack_idx(src_idx1, dst_idx1, CH1)
    p2i0 = pack_idx(src_idx0, dst_idx0, CH2)
    p2i1 = pack_idx(src_idx1, dst_idx1, CH2, dn_off=ND)

    dma = pltpu.SemaphoreType.DMA
    ex0, ex1, dnp0, dnp1 = pl.kernel(
        _sc_phase1,
        out_type=[jax.ShapeDtypeStruct((EPAD, H), f32),
                  jax.ShapeDtypeStruct((EPAD, H), f32),
                  jax.ShapeDtypeStruct((2, ND, D), f32),
                  jax.ShapeDtypeStruct((2, ND, D), f32)],
        mesh=mesh,
        scratch_types=[[pltpu.VMEM((8, CH1), i32) for _ in range(4)],
                       [pltpu.VMEM((CH1, D), f32) for _ in range(2)],
                       [pltpu.VMEM((CH1, D), f32) for _ in range(2)],
                       [pltpu.VMEM((CH1, H), f32) for _ in range(2)],
                       [pltpu.VMEM((CH1, D), f32) for _ in range(2)],
                       pltpu.VMEM_SHARED((ND, D), f32),
                       pltpu.VMEM_SHARED((ND, D), f32),
                       [dma for _ in range(4)], [dma, dma], [dma, dma],
                       [dma, dma], [dma, dma]],
        compiler_params=sc_params,
    )(p1i0, p1i1, qs, kt0, kt1, zn)

    dspec = pl.BlockSpec((2, ND // 5, D), lambda i: (0, i, 0))
    pspec = pl.BlockSpec((ND // 5, D), lambda i: (i, 0))
    rcp0, rcp1 = pl.pallas_call(
        _rcp_body,
        grid=(5,),
        in_specs=[pl.BlockSpec((1, 2), lambda i: (0, 0)), dspec, dspec],
        out_specs=[pspec, pspec],
        out_shape=[jax.ShapeDtypeStruct((ND, D), f32),
                   jax.ShapeDtypeStruct((ND, D), f32)],
    )(gate_logits.reshape(1, 2), dnp0, dnp1)

    hmsgp = pl.kernel(
        _sc_phase2,
        out_type=jax.ShapeDtypeStruct((2, N, D), f32),
        mesh=mesh,
        scratch_types=[[pltpu.VMEM((8, CH2), i32) for _ in range(4)],
                       [pltpu.VMEM((CH2, D), f32) for _ in range(2)],
                       [pltpu.VMEM((CH2, D), f32) for _ in range(2)],
                       [pltpu.VMEM((CH2, H), f32) for _ in range(2)],
                       [pltpu.VMEM((CH2, D), f32) for _ in range(2)],
                       pltpu.VMEM_SHARED((NPAD, D), f32),
                       pltpu.VMEM_SHARED((2 * ND, D), f32),
                       [dma for _ in range(4)], [dma, dma], [dma, dma],
                       [dma, dma], [dma, dma]],
        compiler_params=sc_params,
    )(p2i0, p2i1, v, ex0, ex1, rcp0, rcp1, zn)

    return pl.pallas_call(
        _out_body,
        grid=(_NBLK,),
        in_specs=[pl.BlockSpec((2, _BLK, D), lambda i: (0, i, 0)), nspec,
                  wspec, bspec, bspec, bspec],
        out_specs=nspec,
        out_shape=nshape,
    )(hmsgp, h, Ow, row(Ob), row(ln_s), row(ln_b))


# speedup vs baseline: 1.0557x; 1.0557x over previous
import jax
import jax.numpy as jnp
import numpy as np
from jax import lax
from jax.experimental import pallas as pl
from jax.experimental.pallas import tpu as pltpu
from jax.experimental.pallas import tpu_sc as plsc

N = 10000
E = 160000
D = 128
H = 8
DH = 16
EPS = 1e-8

_BLK = 1000
_NBLK = N // _BLK

NW = 32
EPAD = 163840
PADN = 10008
CH1 = 32
PW1 = EPAD // CH1 // NW
CH2 = 16
PW2 = EPAD // CH2 // NW
ROWS_T = 624
ND = 640
DROWS_T = ND // 16
NPAD = 10016


def _proj_body(h_ref, qw_ref, qb_ref, kw_ref, kb_ref, vw_ref, vb_ref,
               bd0_ref, bd1_ref, qs_ref, kt0_ref, kt1_ref, v_ref):
    hb = h_ref[...]
    k = hb @ kw_ref[...] + kb_ref[...]
    qs_ref[...] = hb @ qw_ref[...] + qb_ref[...]
    kt0_ref[...] = k @ bd0_ref[...]
    kt1_ref[...] = k @ bd1_ref[...]
    v_ref[...] = hb @ vw_ref[...] + vb_ref[...]


def _rcp_body(gl_ref, dn0_ref, dn1_ref, rcp0_ref, rcp1_ref):
    g = jax.nn.sigmoid(gl_ref[...])
    d0 = dn0_ref[...]
    d1 = dn1_ref[...]
    rcp0_ref[...] = g[0, 0] / jnp.maximum(d0[0] + d0[1], EPS)
    rcp1_ref[...] = g[0, 1] / jnp.maximum(d1[0] + d1[1], EPS)


def _out_body(hmsgp_ref, h_ref, ow_ref, ob_ref, lns_ref, lnb_ref, o_ref):
    hm = hmsgp_ref[...]
    t = jax.nn.gelu(hm[0] + hm[1]) @ ow_ref[...] + ob_ref[...]
    x = t + h_ref[...]
    mu = jnp.mean(x, axis=-1, keepdims=True)
    var = jnp.mean((x - mu) ** 2, axis=-1, keepdims=True)
    o_ref[...] = (x - mu) / jnp.sqrt(var + 1e-5) * lns_ref[...] + lnb_ref[...]


def _worker_ids():
    c = lax.axis_index("c")
    s = lax.axis_index("s")
    return c, s, s * 2 + c


def _lanes():
    return lax.iota(jnp.int32, 16)


def _p1_compute(idxb, qg, ktg, exv, ex128):
    def group(g, _):
        rows = g * 16 + _lanes()
        colb = (idxb[1, pl.ds(g * 16, 16)] & 15) * 8
        for h in range(H):
            acc = None
            for dd in range(DH):
                col = jnp.full((16,), h * DH + dd, jnp.int32)
                qv = plsc.load_gather(qg, [rows, col])
                kv = plsc.load_gather(ktg, [rows, col])
                acc = qv * kv if acc is None else acc + qv * kv
            ex = jnp.exp(acc)
            plsc.store_scatter(exv, [rows, jnp.full((16,), h, jnp.int32)], ex)
            plsc.store_scatter(ex128, [rows, colb + h], ex)
        return 0

    lax.fori_loop(0, CH1 // 16, group, 0)


def _p1_rezero(idxb, ex128):
    def group(g, _):
        rows = g * 16 + _lanes()
        colb = (idxb[1, pl.ds(g * 16, 16)] & 15) * 8
        zv = jnp.zeros((16,), jnp.float32)
        for h in range(H):
            plsc.store_scatter(ex128, [rows, colb + h], zv)
        return 0

    lax.fori_loop(0, CH1 // 16, group, 0)


def _sc_phase1(pidx0, pidx1, qs, kt0, kt1, zn,
               ex0, ex1, dnp0, dnp1,
               idxb, qg, ktg, exv, ex128, dn0_sh, dn1_sh,
               sem_i, sem_gq, sem_gk, sem_s, sem_w):
    sc, tid, wid = _worker_ids()
    drb = tid * DROWS_T
    stg = ex128[0].at[pl.ds(0, 16)]

    def zchunk(j, _):
        rb = drb + j * 8
        s8 = ex128[0].at[pl.ds(0, 8)]
        pltpu.sync_copy(zn.at[pl.ds(rb, 8)], s8)
        pltpu.sync_copy(s8, dn0_sh.at[pl.ds(rb, 8)])
        pltpu.sync_copy(s8, dn1_sh.at[pl.ds(rb, 8)])
        return 0

    lax.fori_loop(0, DROWS_T // 8, zchunk, 0)
    plsc.subcore_barrier()

    cb0 = wid * PW1

    for pidx, kt, exh, dnsh in ((pidx0, kt0, ex0, dn0_sh),
                                (pidx1, kt1, ex1, dn1_sh)):
        pltpu.sync_copy(zn.at[pl.ds(0, CH1)], ex128[0])
        pltpu.sync_copy(zn.at[pl.ds(0, CH1)], ex128[1])
        for r in range(3):
            pltpu.async_copy(pidx.at[cb0 + r], idxb[r], sem_i[r])
        pltpu.make_async_copy(pidx.at[cb0], idxb[0], sem_i[0]).wait()
        pltpu.async_copy(qs.at[idxb[0].at[1]], qg[0], sem_gq[0])
        pltpu.async_copy(kt.at[idxb[0].at[0]], ktg[0], sem_gk[0])

        def quad(j, _, pidx=pidx, kt=kt, exh=exh, dnsh=dnsh):
            for r in range(4):
                i = j * 4 + r
                b = r % 2
                rn = (r + 1) % 4
                ro = (r - 1) % 4

                @pl.when(i + 1 < PW1)
                def _(i=i, b=b, rn=rn):
                    pltpu.make_async_copy(pidx.at[cb0 + i + 1], idxb[rn],
                                          sem_i[rn]).wait()
                    pltpu.async_copy(qs.at[idxb[rn].at[1]], qg[1 - b],
                                     sem_gq[1 - b])
                    pltpu.async_copy(kt.at[idxb[rn].at[0]], ktg[1 - b],
                                     sem_gk[1 - b])

                pltpu.make_async_copy(qs.at[idxb[r].at[1]], qg[b],
                                      sem_gq[b]).wait()
                pltpu.make_async_copy(kt.at[idxb[r].at[0]], ktg[b],
                                      sem_gk[b]).wait()
                _p1_compute(idxb[r], qg[b], ktg[b], exv[b], ex128[b])
                pltpu.async_copy(ex128[b], dnsh.at[idxb[r].at[2]], sem_s[b],
                                 add=True)
                pltpu.async_copy(exv[b], exh.at[pl.ds((cb0 + i) * CH1, CH1)],
                                 sem_w[b])

                @pl.when(i >= 1)
                def _(i=i, b=b, ro=ro, dnsh=dnsh, exh=exh):
                    pltpu.make_async_copy(ex128[1 - b], dnsh.at[idxb[ro].at[2]],
                                          sem_s[1 - b]).wait()
                    pltpu.make_async_copy(
                        exv[1 - b], exh.at[pl.ds((cb0 + i - 1) * CH1, CH1)],
                        sem_w[1 - b]).wait()
                    _p1_rezero(idxb[ro], ex128[1 - b])

                @pl.when(i + 3 < PW1)
                def _(i=i, ro=ro, pidx=pidx):
                    pltpu.async_copy(pidx.at[cb0 + i + 3], idxb[ro], sem_i[ro])
            return 0

        lax.fori_loop(0, PW1 // 4, quad, 0)
        bl = (PW1 - 1) % 2
        rl = (PW1 - 1) % 4
        pltpu.make_async_copy(ex128[bl], dnsh.at[idxb[rl].at[2]],
                              sem_s[bl]).wait()
        pltpu.make_async_copy(exv[bl], exh.at[pl.ds((cb0 + PW1 - 1) * CH1, CH1)],
                              sem_w[bl]).wait()
        _p1_rezero(idxb[rl], ex128[bl])

    plsc.subcore_barrier()

    def rchunk(j, _):
        rb = drb + j * 8
        s8 = ex128[0].at[pl.ds(0, 8)]
        pltpu.sync_copy(dn0_sh.at[pl.ds(rb, 8)], s8)
        pltpu.sync_copy(s8, dnp0.at[sc, pl.ds(rb, 8)])
        pltpu.sync_copy(dn1_sh.at[pl.ds(rb, 8)], s8)
        pltpu.sync_copy(s8, dnp1.at[sc, pl.ds(rb, 8)])
        return 0

    lax.fori_loop(0, DROWS_T // 8, rchunk, 0)


def _p2_compute(idxb, vg, rcpg, exv, msgv):
    def group(g, _):
        rows = g * 16 + _lanes()
        colb = (idxb[1, pl.ds(g * 16, 16)] & 15) * 8
        for h in range(H):
            aw = (plsc.load_gather(exv, [rows, jnp.full((16,), h, jnp.int32)]) *
                  plsc.load_gather(rcpg, [rows, colb + h]))
            for dd in range(DH):
                col = jnp.full((16,), h * DH + dd, jnp.int32)
                mv = plsc.load_gather(vg, [rows, col]) * aw
                plsc.store_scatter(msgv, [rows, col], mv)
        return 0

    lax.fori_loop(0, CH2 // 16, group, 0)


def _sc_phase2(pidx0, pidx1, v, ex0, ex1, rcp0, rcp1, zn,
               hmsgp,
               idxb, vg, rcpg, exv, msgv, hmsg_sh,
               sem_i, sem_gv, sem_gr, sem_ge, sem_s):
    sc, tid, wid = _worker_ids()
    rbase = tid * ROWS_T
    nzc = jnp.where(tid == 15, 40, 39)
    stg = msgv[0].at[pl.ds(0, 16)]

    def zchunk(j, _):
        rb = rbase + j * 16
        pltpu.sync_copy(zn.at[pl.ds(rb, 16)], stg)
        pltpu.sync_copy(stg, hmsg_sh.at[pl.ds(rb, 16)])
        return 0

    lax.fori_loop(0, nzc, zchunk, 0)
    plsc.subcore_barrier()

    cb0 = wid * PW2

    for pidx, exh, rcp in ((pidx0, ex0, rcp0), (pidx1, ex1, rcp1)):
        for r in range(3):
            pltpu.async_copy(pidx.at[cb0 + r], idxb[r], sem_i[r])
        pltpu.make_async_copy(pidx.at[cb0], idxb[0], sem_i[0]).wait()
        pltpu.async_copy(v.at[idxb[0].at[0]], vg[0], sem_gv[0])
        pltpu.async_copy(rcp.at[idxb[0].at[2]], rcpg[0], sem_gr[0])
        pltpu.async_copy(exh.at[pl.ds(cb0 * CH2, CH2)], exv[0], sem_ge[0])

        def quad(j, _, pidx=pidx, exh=exh, rcp=rcp):
            for r in range(4):
                i = j * 4 + r
                b = r % 2
                rn = (r + 1) % 4
                ro = (r - 1) % 4

                @pl.when(i + 1 < PW2)
                def _(i=i, b=b, rn=rn, rcp=rcp, exh=exh):
                    pltpu.make_async_copy(pidx.at[cb0 + i + 1], idxb[rn],
                                          sem_i[rn]).wait()
                    pltpu.async_copy(v.at[idxb[rn].at[0]], vg[1 - b],
                                     sem_gv[1 - b])
                    pltpu.async_copy(rcp.at[idxb[rn].at[2]], rcpg[1 - b],
                                     sem_gr[1 - b])
                    pltpu.async_copy(exh.at[pl.ds((cb0 + i + 1) * CH2, CH2)],
                                     exv[1 - b], sem_ge[1 - b])

                pltpu.make_async_copy(v.at[idxb[r].at[0]], vg[b],
                                      sem_gv[b]).wait()
                pltpu.make_async_copy(rcp.at[idxb[r].at[2]], rcpg[b],
                                      sem_gr[b]).wait()
                pltpu.make_async_copy(exh.at[pl.ds((cb0 + i) * CH2, CH2)],
                                      exv[b], sem_ge[b]).wait()
                _p2_compute(idxb[r], vg[b], rcpg[b], exv[b], msgv[b])
                pltpu.async_copy(msgv[b], hmsg_sh.at[idxb[r].at[3]], sem_s[b],
                                 add=True)

                @pl.when(i >= 1)
                def _(i=i, b=b, ro=ro):
                    pltpu.make_async_copy(msgv[1 - b],
                                          hmsg_sh.at[idxb[ro].at[3]],
                                          sem_s[1 - b]).wait()

                @pl.when(i + 3 < PW2)
                def _(i=i, ro=ro, pidx=pidx):
                    pltpu.async_copy(pidx.at[cb0 + i + 3], idxb[ro], sem_i[ro])
            return 0

        lax.fori_loop(0, PW2 // 4, quad, 0)
        bl = (PW2 - 1) % 2
        rl = (PW2 - 1) % 4
        pltpu.make_async_copy(msgv[bl], hmsg_sh.at[idxb[rl].at[3]],
                              sem_s[bl]).wait()

    plsc.subcore_barrier()

    def rchunk(j, _):
        rb = rbase + j * 16
        pltpu.sync_copy(hmsg_sh.at[pl.ds(rb, 16)], stg)
        pltpu.sync_copy(stg, hmsgp.at[sc, pl.ds(rb, 16)])
        return 0

    lax.fori_loop(0, nzc, rchunk, 0)


def kernel(h, src_idx0, dst_idx0, src_idx1, dst_idx1, Qw, Qb, Kw, Kb, Vw, Vb,
           edge_W, gate_logits, Ow, Ob, ln_s, ln_b):
    scale = 1.0 / np.sqrt(DH)
    bd = jnp.zeros((2, H, DH, H, DH), jnp.float32)
    bd = bd.at[:, jnp.arange(H), :, jnp.arange(H), :].set(
        jnp.transpose(edge_W, (1, 0, 2, 3))).reshape(2, D, D)

    row = lambda b: b.reshape(1, D)
    wspec = pl.BlockSpec((D, D), lambda i: (0, 0))
    bspec = pl.BlockSpec((1, D), lambda i: (0, 0))
    nspec = pl.BlockSpec((_BLK, D), lambda i: (i, 0))
    nshape = jax.ShapeDtypeStruct((N, D), jnp.float32)

    qs, kt0, kt1, v = pl.pallas_call(
        _proj_body,
        grid=(_NBLK,),
        in_specs=[nspec, wspec, bspec, wspec, bspec, wspec, bspec, wspec, wspec],
        out_specs=[nspec, nspec, nspec, nspec],
        out_shape=[nshape, nshape, nshape, nshape],
    )(h, Qw * scale, row(Qb) * scale, Kw, row(Kb), Vw, row(Vb), bd[0], bd[1])

    mesh = plsc.VectorSubcoreMesh(core_axis_name="c", subcore_axis_name="s")
    f32 = jnp.float32
    i32 = jnp.int32
    sc_params = pltpu.CompilerParams(needs_layout_passes=False)
    zn = jnp.zeros((N, D), f32)

    npad = EPAD - E

    def pack_idx(si, di, ch, dn_off=0):
        zp = jnp.zeros((npad,), i32)
        si_p = jnp.concatenate([si, zp])
        dig = jnp.concatenate([di, zp])
        dnr = jnp.concatenate([lax.shift_right_logical(di, 4) + dn_off,
                               jnp.full((npad,), dn_off + ND - 1, i32)])
        hmr = jnp.concatenate([di, jnp.full((npad,), PADN, i32)])
        nch = EPAD // ch
        z = jnp.zeros((nch, ch), i32)
        return jnp.stack([a.reshape(nch, ch) for a in (si_p, dig, dnr, hmr)] +
                         [z, z, z, z], axis=1)

    p1i0 = pack_idx(src_idx0, dst_idx0, CH1)
    p1i1 = pack_idx(src_idx1, dst_idx1, CH1)
    p2i0 = pack_idx(src_idx0, dst_idx0, CH2)
    p2i1 = pack_idx(src_idx1, dst_idx1, CH2)

    dma = pltpu.SemaphoreType.DMA
    ex0, ex1, dnp0, dnp1 = pl.kernel(
        _sc_phase1,
        out_type=[jax.ShapeDtypeStruct((EPAD, H), f32),
                  jax.ShapeDtypeStruct((EPAD, H), f32),
                  jax.ShapeDtypeStruct((2, ND, D), f32),
                  jax.ShapeDtypeStruct((2, ND, D), f32)],
        mesh=mesh,
        scratch_types=[[pltpu.VMEM((8, CH1), i32) for _ in range(4)],
                       [pltpu.VMEM((CH1, D), f32) for _ in range(2)],
                       [pltpu.VMEM((CH1, D), f32) for _ in range(2)],
                       [pltpu.VMEM((CH1, H), f32) for _ in range(2)],
                       [pltpu.VMEM((CH1, D), f32) for _ in range(2)],
                       pltpu.VMEM_SHARED((ND, D), f32),
                       pltpu.VMEM_SHARED((ND, D), f32),
                       [dma for _ in range(4)], [dma, dma], [dma, dma],
                       [dma, dma], [dma, dma]],
        compiler_params=sc_params,
    )(p1i0, p1i1, qs, kt0, kt1, zn)

    dspec = pl.BlockSpec((2, ND // 5, D), lambda i: (0, i, 0))
    pspec = pl.BlockSpec((ND // 5, D), lambda i: (i, 0))
    rcp0, rcp1 = pl.pallas_call(
        _rcp_body,
        grid=(5,),
        in_specs=[pl.BlockSpec((1, 2), lambda i: (0, 0)), dspec, dspec],
        out_specs=[pspec, pspec],
        out_shape=[jax.ShapeDtypeStruct((ND, D), f32),
                   jax.ShapeDtypeStruct((ND, D), f32)],
    )(gate_logits.reshape(1, 2), dnp0, dnp1)

    hmsgp = pl.kernel(
        _sc_phase2,
        out_type=jax.ShapeDtypeStruct((2, N, D), f32),
        mesh=mesh,
        scratch_types=[[pltpu.VMEM((8, CH2), i32) for _ in range(4)],
                       [pltpu.VMEM((CH2, D), f32) for _ in range(2)],
                       [pltpu.VMEM((CH2, D), f32) for _ in range(2)],
                       [pltpu.VMEM((CH2, H), f32) for _ in range(2)],
                       [pltpu.VMEM((CH2, D), f32) for _ in range(2)],
                       pltpu.VMEM_SHARED((NPAD, D), f32),
                       [dma for _ in range(4)], [dma, dma], [dma, dma],
                       [dma, dma], [dma, dma]],
        compiler_params=sc_params,
    )(p2i0, p2i1, v, ex0, ex1, rcp0, rcp1, zn)

    return pl.pallas_call(
        _out_body,
        grid=(_NBLK,),
        in_specs=[pl.BlockSpec((2, _BLK, D), lambda i: (0, i, 0)), nspec,
                  wspec, bspec, bspec, bspec],
        out_specs=nspec,
        out_shape=nshape,
    )(hmsgp, h, Ow, row(Ob), row(ln_s), row(ln_b))


# speedup vs baseline: 1.0563x; 1.0006x over previous
import jax
import jax.numpy as jnp
import numpy as np
from jax import lax
from jax.experimental import pallas as pl
from jax.experimental.pallas import tpu as pltpu
from jax.experimental.pallas import tpu_sc as plsc

N = 10000
E = 160000
D = 128
H = 8
DH = 16
EPS = 1e-8

_BLK = 1000
_NBLK = N // _BLK

NW = 32
EPAD = 163840
PADN = 10008
CH1 = 32
PW1 = EPAD // CH1 // NW
CH2 = 16
PW2 = EPAD // CH2 // NW
ROWS_T = 624
ND = 640
DROWS_T = ND // 16
NPAD = 10016


def _proj_body(h_ref, qw_ref, qb_ref, kw_ref, kb_ref, vw_ref, vb_ref,
               bd0_ref, bd1_ref, qs_ref, kt0_ref, kt1_ref, v_ref):
    hb = h_ref[...]
    k = hb @ kw_ref[...] + kb_ref[...]
    qs_ref[...] = hb @ qw_ref[...] + qb_ref[...]
    kt0_ref[...] = k @ bd0_ref[...]
    kt1_ref[...] = k @ bd1_ref[...]
    v_ref[...] = hb @ vw_ref[...] + vb_ref[...]


def _rcp_body(gl_ref, dn0_ref, dn1_ref, rcp0_ref, rcp1_ref):
    g = jax.nn.sigmoid(gl_ref[...])
    d0 = dn0_ref[...]
    d1 = dn1_ref[...]
    rcp0_ref[...] = g[0, 0] / jnp.maximum(d0[0] + d0[1], EPS)
    rcp1_ref[...] = g[0, 1] / jnp.maximum(d1[0] + d1[1], EPS)


def _out_body(hmsgp_ref, h_ref, ow_ref, ob_ref, lns_ref, lnb_ref, o_ref):
    hm = hmsgp_ref[...]
    t = jax.nn.gelu(hm[0] + hm[1]) @ ow_ref[...] + ob_ref[...]
    x = t + h_ref[...]
    mu = jnp.mean(x, axis=-1, keepdims=True)
    var = jnp.mean((x - mu) ** 2, axis=-1, keepdims=True)
    o_ref[...] = (x - mu) / jnp.sqrt(var + 1e-5) * lns_ref[...] + lnb_ref[...]


def _worker_ids():
    c = lax.axis_index("c")
    s = lax.axis_index("s")
    return c, s, s * 2 + c


def _lanes():
    return lax.iota(jnp.int32, 16)


def _p1_compute(idxb, qg, ktg, exv, ex128):
    def group(g, _):
        rows = g * 16 + _lanes()
        colb = (idxb[1, pl.ds(g * 16, 16)] & 15) * 8
        for h in range(H):
            acc = None
            for dd in range(DH):
                col = jnp.full((16,), h * DH + dd, jnp.int32)
                qv = plsc.load_gather(qg, [rows, col])
                kv = plsc.load_gather(ktg, [rows, col])
                acc = qv * kv if acc is None else acc + qv * kv
            ex = jnp.exp(acc)
            plsc.store_scatter(exv, [rows, jnp.full((16,), h, jnp.int32)], ex)
            plsc.store_scatter(ex128, [rows, colb + h], ex)
        return 0

    lax.fori_loop(0, CH1 // 16, group, 0)


def _p1_rezero(idxb, ex128):
    def group(g, _):
        rows = g * 16 + _lanes()
        colb = (idxb[1, pl.ds(g * 16, 16)] & 15) * 8
        zv = jnp.zeros((16,), jnp.float32)
        for h in range(H):
            plsc.store_scatter(ex128, [rows, colb + h], zv)
        return 0

    lax.fori_loop(0, CH1 // 16, group, 0)


def _sc_phase1(pidx0, pidx1, qs, kt0, kt1, zn,
               ex0, ex1, dnp0, dnp1,
               idxb, qg, ktg, exv, ex128, dn0_sh, dn1_sh,
               sem_i, sem_gq, sem_gk, sem_s, sem_w):
    sc, tid, wid = _worker_ids()
    drb = tid * DROWS_T
    stg = ex128[0].at[pl.ds(0, 16)]

    def zchunk(j, _):
        rb = drb + j * 8
        s8 = ex128[0].at[pl.ds(0, 8)]
        pltpu.sync_copy(zn.at[pl.ds(rb, 8)], s8)
        pltpu.sync_copy(s8, dn0_sh.at[pl.ds(rb, 8)])
        pltpu.sync_copy(s8, dn1_sh.at[pl.ds(rb, 8)])
        return 0

    lax.fori_loop(0, DROWS_T // 8, zchunk, 0)
    plsc.subcore_barrier()

    cb0 = wid * PW1

    for pidx, kt, exh, dnsh in ((pidx0, kt0, ex0, dn0_sh),
                                (pidx1, kt1, ex1, dn1_sh)):
        pltpu.sync_copy(zn.at[pl.ds(0, CH1)], ex128[0])
        pltpu.sync_copy(zn.at[pl.ds(0, CH1)], ex128[1])
        for r in range(3):
            pltpu.async_copy(pidx.at[cb0 + r], idxb[r], sem_i[r])
        pltpu.make_async_copy(pidx.at[cb0], idxb[0], sem_i[0]).wait()
        pltpu.async_copy(qs.at[idxb[0].at[1]], qg[0], sem_gq[0])
        pltpu.async_copy(kt.at[idxb[0].at[0]], ktg[0], sem_gk[0])

        def quad(j, _, pidx=pidx, kt=kt, exh=exh, dnsh=dnsh):
            for r in range(4):
                i = j * 4 + r
                b = r % 2
                rn = (r + 1) % 4
                ro = (r - 1) % 4

                @pl.when(i + 1 < PW1)
                def _(i=i, b=b, rn=rn):
                    pltpu.make_async_copy(pidx.at[cb0 + i + 1], idxb[rn],
                                          sem_i[rn]).wait()
                    pltpu.async_copy(qs.at[idxb[rn].at[1]], qg[1 - b],
                                     sem_gq[1 - b])
                    pltpu.async_copy(kt.at[idxb[rn].at[0]], ktg[1 - b],
                                     sem_gk[1 - b])

                pltpu.make_async_copy(qs.at[idxb[r].at[1]], qg[b],
                                      sem_gq[b]).wait()
                pltpu.make_async_copy(kt.at[idxb[r].at[0]], ktg[b],
                                      sem_gk[b]).wait()
                _p1_compute(idxb[r], qg[b], ktg[b], exv[b], ex128[b])
                pltpu.async_copy(ex128[b], dnsh.at[idxb[r].at[2]], sem_s[b],
                                 add=True)
                pltpu.async_copy(exv[b], exh.at[pl.ds((cb0 + i) * CH1, CH1)],
                                 sem_w[b])

                @pl.when(i >= 1)
                def _(i=i, b=b, ro=ro, dnsh=dnsh, exh=exh):
                    pltpu.make_async_copy(ex128[1 - b], dnsh.at[idxb[ro].at[2]],
                                          sem_s[1 - b]).wait()
                    pltpu.make_async_copy(
                        exv[1 - b], exh.at[pl.ds((cb0 + i - 1) * CH1, CH1)],
                        sem_w[1 - b]).wait()
                    _p1_rezero(idxb[ro], ex128[1 - b])

                @pl.when(i + 3 < PW1)
                def _(i=i, ro=ro, pidx=pidx):
                    pltpu.async_copy(pidx.at[cb0 + i + 3], idxb[ro], sem_i[ro])
            return 0

        lax.fori_loop(0, PW1 // 4, quad, 0)
        bl = (PW1 - 1) % 2
        rl = (PW1 - 1) % 4
        pltpu.make_async_copy(ex128[bl], dnsh.at[idxb[rl].at[2]],
                              sem_s[bl]).wait()
        pltpu.make_async_copy(exv[bl], exh.at[pl.ds((cb0 + PW1 - 1) * CH1, CH1)],
                              sem_w[bl]).wait()
        _p1_rezero(idxb[rl], ex128[bl])

    plsc.subcore_barrier()

    def rchunk(j, _):
        rb = drb + j * 8
        s8 = ex128[0].at[pl.ds(0, 8)]
        pltpu.sync_copy(dn0_sh.at[pl.ds(rb, 8)], s8)
        pltpu.sync_copy(s8, dnp0.at[sc, pl.ds(rb, 8)])
        pltpu.sync_copy(dn1_sh.at[pl.ds(rb, 8)], s8)
        pltpu.sync_copy(s8, dnp1.at[sc, pl.ds(rb, 8)])
        return 0

    lax.fori_loop(0, DROWS_T // 8, rchunk, 0)


def _p2_compute(idxb, vrcpg, exv, msgv):
    def group(g, _):
        rows = g * 16 + _lanes()
        colb = (idxb[1, pl.ds(g * 16, 16)] & 15) * 8
        for h in range(H):
            aw = (plsc.load_gather(exv, [rows, jnp.full((16,), h, jnp.int32)]) *
                  plsc.load_gather(vrcpg, [rows + CH2, colb + h]))
            for dd in range(DH):
                col = jnp.full((16,), h * DH + dd, jnp.int32)
                mv = plsc.load_gather(vrcpg, [rows, col]) * aw
                plsc.store_scatter(msgv, [rows, col], mv)
        return 0

    lax.fori_loop(0, CH2 // 16, group, 0)


def _fill_gx(idxb, gx):
    gx[pl.ds(0, 16)] = idxb[4, pl.ds(0, 16)]
    gx[pl.ds(16, 16)] = idxb[5, pl.ds(0, 16)]


def _sc_phase2(pidx0, pidx1, vrcp, ex0, ex1, zn,
               hmsgp,
               idxb, gx, vrcpg, exv, msgv, hmsg_sh,
               sem_i, sem_gv, sem_ge, sem_s):
    sc, tid, wid = _worker_ids()
    rbase = tid * ROWS_T
    nzc = jnp.where(tid == 15, 40, 39)
    stg = msgv[0].at[pl.ds(0, 16)]

    def zchunk(j, _):
        rb = rbase + j * 16
        pltpu.sync_copy(zn.at[pl.ds(rb, 16)], stg)
        pltpu.sync_copy(stg, hmsg_sh.at[pl.ds(rb, 16)])
        return 0

    lax.fori_loop(0, nzc, zchunk, 0)
    plsc.subcore_barrier()

    cb0 = wid * PW2

    for pidx, exh in ((pidx0, ex0), (pidx1, ex1)):
        for r in range(3):
            pltpu.async_copy(pidx.at[cb0 + r], idxb[r], sem_i[r])
        pltpu.make_async_copy(pidx.at[cb0], idxb[0], sem_i[0]).wait()
        _fill_gx(idxb[0], gx[0])
        pltpu.async_copy(vrcp.at[gx[0]], vrcpg[0], sem_gv[0])
        pltpu.async_copy(exh.at[pl.ds(cb0 * CH2, CH2)], exv[0], sem_ge[0])

        def quad(j, _, pidx=pidx, exh=exh):
            for r in range(4):
                i = j * 4 + r
                b = r % 2
                rn = (r + 1) % 4
                ro = (r - 1) % 4

                @pl.when(i + 1 < PW2)
                def _(i=i, b=b, rn=rn, exh=exh):
                    pltpu.make_async_copy(pidx.at[cb0 + i + 1], idxb[rn],
                                          sem_i[rn]).wait()
                    _fill_gx(idxb[rn], gx[1 - b])
                    pltpu.async_copy(vrcp.at[gx[1 - b]], vrcpg[1 - b],
                                     sem_gv[1 - b])
                    pltpu.async_copy(exh.at[pl.ds((cb0 + i + 1) * CH2, CH2)],
                                     exv[1 - b], sem_ge[1 - b])

                pltpu.make_async_copy(vrcp.at[gx[b]], vrcpg[b],
                                      sem_gv[b]).wait()
                pltpu.make_async_copy(exh.at[pl.ds((cb0 + i) * CH2, CH2)],
                                      exv[b], sem_ge[b]).wait()
                _p2_compute(idxb[r], vrcpg[b], exv[b], msgv[b])
                pltpu.async_copy(msgv[b], hmsg_sh.at[idxb[r].at[3]], sem_s[b],
                                 add=True)

                @pl.when(i >= 1)
                def _(i=i, b=b, ro=ro):
                    pltpu.make_async_copy(msgv[1 - b],
                                          hmsg_sh.at[idxb[ro].at[3]],
                                          sem_s[1 - b]).wait()

                @pl.when(i + 3 < PW2)
                def _(i=i, ro=ro, pidx=pidx):
                    pltpu.async_copy(pidx.at[cb0 + i + 3], idxb[ro], sem_i[ro])
            return 0

        lax.fori_loop(0, PW2 // 4, quad, 0)
        bl = (PW2 - 1) % 2
        rl = (PW2 - 1) % 4
        pltpu.make_async_copy(msgv[bl], hmsg_sh.at[idxb[rl].at[3]],
                              sem_s[bl]).wait()

    plsc.subcore_barrier()

    def rchunk(j, _):
        rb = rbase + j * 16
        pltpu.sync_copy(hmsg_sh.at[pl.ds(rb, 16)], stg)
        pltpu.sync_copy(stg, hmsgp.at[sc, pl.ds(rb, 16)])
        return 0

    lax.fori_loop(0, nzc, rchunk, 0)


def kernel(h, src_idx0, dst_idx0, src_idx1, dst_idx1, Qw, Qb, Kw, Kb, Vw, Vb,
           edge_W, gate_logits, Ow, Ob, ln_s, ln_b):
    scale = 1.0 / np.sqrt(DH)
    bd = jnp.zeros((2, H, DH, H, DH), jnp.float32)
    bd = bd.at[:, jnp.arange(H), :, jnp.arange(H), :].set(
        jnp.transpose(edge_W, (1, 0, 2, 3))).reshape(2, D, D)

    row = lambda b: b.reshape(1, D)
    wspec = pl.BlockSpec((D, D), lambda i: (0, 0))
    bspec = pl.BlockSpec((1, D), lambda i: (0, 0))
    nspec = pl.BlockSpec((_BLK, D), lambda i: (i, 0))
    nshape = jax.ShapeDtypeStruct((N, D), jnp.float32)

    qs, kt0, kt1, v = pl.pallas_call(
        _proj_body,
        grid=(_NBLK,),
        in_specs=[nspec, wspec, bspec, wspec, bspec, wspec, bspec, wspec, wspec],
        out_specs=[nspec, nspec, nspec, nspec],
        out_shape=[nshape, nshape, nshape, nshape],
    )(h, Qw * scale, row(Qb) * scale, Kw, row(Kb), Vw, row(Vb), bd[0], bd[1])

    mesh = plsc.VectorSubcoreMesh(core_axis_name="c", subcore_axis_name="s")
    f32 = jnp.float32
    i32 = jnp.int32
    sc_params = pltpu.CompilerParams(needs_layout_passes=False)
    zn = jnp.zeros((N, D), f32)

    npad = EPAD - E

    def pack_idx(si, di, ch, rcp_off=0):
        zp = jnp.zeros((npad,), i32)
        si_p = jnp.concatenate([si, zp])
        dig = jnp.concatenate([di, zp])
        dnr = jnp.concatenate([lax.shift_right_logical(di, 4),
                               jnp.full((npad,), ND - 1, i32)])
        hmr = jnp.concatenate([di, jnp.full((npad,), PADN, i32)])
        nch = EPAD // ch
        z = jnp.zeros((nch, ch), i32)
        return jnp.stack([a.reshape(nch, ch) for a in
                          (si_p, dig, dnr, hmr, si_p, dnr + rcp_off)] +
                         [z, z], axis=1)

    p1i0 = pack_idx(src_idx0, dst_idx0, CH1)
    p1i1 = pack_idx(src_idx1, dst_idx1, CH1)
    p2i0 = pack_idx(src_idx0, dst_idx0, CH2, rcp_off=N)
    p2i1 = pack_idx(src_idx1, dst_idx1, CH2, rcp_off=N + ND)

    dma = pltpu.SemaphoreType.DMA
    ex0, ex1, dnp0, dnp1 = pl.kernel(
        _sc_phase1,
        out_type=[jax.ShapeDtypeStruct((EPAD, H), f32),
                  jax.ShapeDtypeStruct((EPAD, H), f32),
                  jax.ShapeDtypeStruct((2, ND, D), f32),
                  jax.ShapeDtypeStruct((2, ND, D), f32)],
        mesh=mesh,
        scratch_types=[[pltpu.VMEM((8, CH1), i32) for _ in range(4)],
                       [pltpu.VMEM((CH1, D), f32) for _ in range(2)],
                       [pltpu.VMEM((CH1, D), f32) for _ in range(2)],
                       [pltpu.VMEM((CH1, H), f32) for _ in range(2)],
                       [pltpu.VMEM((CH1, D), f32) for _ in range(2)],
                       pltpu.VMEM_SHARED((ND, D), f32),
                       pltpu.VMEM_SHARED((ND, D), f32),
                       [dma for _ in range(4)], [dma, dma], [dma, dma],
                       [dma, dma], [dma, dma]],
        compiler_params=sc_params,
    )(p1i0, p1i1, qs, kt0, kt1, zn)

    dspec = pl.BlockSpec((2, ND // 5, D), lambda i: (0, i, 0))
    pspec = pl.BlockSpec((ND // 5, D), lambda i: (i, 0))
    rcp0, rcp1 = pl.pallas_call(
        _rcp_body,
        grid=(5,),
        in_specs=[pl.BlockSpec((1, 2), lambda i: (0, 0)), dspec, dspec],
        out_specs=[pspec, pspec],
        out_shape=[jax.ShapeDtypeStruct((ND, D), f32),
                   jax.ShapeDtypeStruct((ND, D), f32)],
    )(gate_logits.reshape(1, 2), dnp0, dnp1)

    vrcp = jnp.concatenate([v, rcp0, rcp1], axis=0)
    hmsgp = pl.kernel(
        _sc_phase2,
        out_type=jax.ShapeDtypeStruct((2, N, D), f32),
        mesh=mesh,
        scratch_types=[[pltpu.VMEM((8, CH2), i32) for _ in range(4)],
                       [pltpu.VMEM((2 * CH2,), i32) for _ in range(2)],
                       [pltpu.VMEM((2 * CH2, D), f32) for _ in range(2)],
                       [pltpu.VMEM((CH2, H), f32) for _ in range(2)],
                       [pltpu.VMEM((CH2, D), f32) for _ in range(2)],
                       pltpu.VMEM_SHARED((NPAD, D), f32),
                       [dma for _ in range(4)], [dma, dma], [dma, dma],
                       [dma, dma]],
        compiler_params=sc_params,
    )(p2i0, p2i1, vrcp, ex0, ex1, zn)

    return pl.pallas_call(
        _out_body,
        grid=(_NBLK,),
        in_specs=[pl.BlockSpec((2, _BLK, D), lambda i: (0, i, 0)), nspec,
                  wspec, bspec, bspec, bspec],
        out_specs=nspec,
        out_shape=nshape,
    )(hmsgp, h, Ow, row(Ob), row(ln_s), row(ln_b))
